# baseline (speedup 1.0000x reference)
import jax
import jax.numpy as jnp
from jax import lax
from jax.experimental import pallas as pl
from jax.experimental.pallas import tpu as pltpu
from jax.experimental.pallas import tpu_sc as plsc

_N = 10000
_E = 320000
_F = 128
_NC = 2
_NS = 16
_NW = _NC * _NS
_EW = _E // _NW
_C = 80
_NCHUNK = _EW // _C
_NPAD = 10240
_ZROWS = _NPAD // _NS


def _sc_agg_body(src_hbm, dst_hbm, xa_hbm, zero_hbm, out_hbm,
                 src_v, dst_v, rows0, rows1, sem0, sem1, ssem0, ssem1, acc_sh):
    c = lax.axis_index("c")
    s = lax.axis_index("s")
    w = c * _NS + s

    pltpu.sync_copy(zero_hbm, acc_sh.at[pl.ds(s * _ZROWS, _ZROWS)])
    pltpu.sync_copy(src_hbm.at[w], src_v)
    pltpu.sync_copy(dst_hbm.at[w], dst_v)
    plsc.subcore_barrier()

    pltpu.async_copy(xa_hbm.at[src_v.at[pl.ds(0, _C)]], rows0, sem0)
    pltpu.async_copy(xa_hbm.at[src_v.at[pl.ds(_C, _C)]], rows1, sem1)

    def step2(jj, carry):
        j0 = jj * 2
        pltpu.make_async_copy(xa_hbm.at[src_v.at[pl.ds(j0 * _C, _C)]], rows0, sem0).wait()
        pltpu.async_copy(rows0, acc_sh.at[dst_v.at[j0]], ssem0, add=True)
        pltpu.make_async_copy(xa_hbm.at[src_v.at[pl.ds((j0 + 1) * _C, _C)]], rows1, sem1).wait()
        pltpu.async_copy(rows1, acc_sh.at[dst_v.at[j0 + 1]], ssem1, add=True)
        pltpu.make_async_copy(rows0, acc_sh.at[dst_v.at[j0]], ssem0).wait()
        pltpu.async_copy(xa_hbm.at[src_v.at[pl.ds((j0 + 2) * _C, _C)]], rows0, sem0)
        pltpu.make_async_copy(rows1, acc_sh.at[dst_v.at[j0 + 1]], ssem1).wait()

        @pl.when(jj < _NCHUNK // 2 - 1)
        def _():
            pltpu.async_copy(xa_hbm.at[src_v.at[pl.ds((j0 + 3) * _C, _C)]], rows1, sem1)

        return carry

    lax.fori_loop(0, _NCHUNK // 2, step2, 0)
    pltpu.make_async_copy(xa_hbm.at[src_v.at[pl.ds((_NCHUNK - 1) * _C, _C)]], rows0, sem0).wait()
    pltpu.sync_copy(rows0, acc_sh.at[dst_v.at[_NCHUNK - 1]], add=True)

    plsc.subcore_barrier()
    pltpu.sync_copy(acc_sh.at[pl.ds(s * _ZROWS, _ZROWS)],
                    out_hbm.at[c, pl.ds(s * _ZROWS, _ZROWS)])


def _sc_agg(xa, src3, dst3, zeros):
    mesh = plsc.VectorSubcoreMesh(core_axis_name="c", subcore_axis_name="s")
    return pl.kernel(
        _sc_agg_body,
        out_type=jax.ShapeDtypeStruct((_NC, _NPAD, _F), jnp.float32),
        mesh=mesh,
        scratch_types=[
            pltpu.VMEM((_EW,), jnp.int32),
            pltpu.VMEM((_NCHUNK, _C), jnp.int32),
            pltpu.VMEM((_C, _F), jnp.float32),
            pltpu.VMEM((_C, _F), jnp.float32),
            pltpu.SemaphoreType.DMA,
            pltpu.SemaphoreType.DMA,
            pltpu.SemaphoreType.DMA,
            pltpu.SemaphoreType.DMA,
            pltpu.VMEM_SHARED((_NPAD, _F), jnp.float32),
        ],
    )(src3, dst3, xa, zeros)


def _sc_cnt_body(dst_hbm, zero_hbm, ones_hbm, outc_hbm,
                 dst_v, ones_v, acc_sh):
    c = lax.axis_index("c")
    s = lax.axis_index("s")
    w = c * _NS + s

    pltpu.sync_copy(zero_hbm, acc_sh.at[pl.ds(s * _ZROWS, _ZROWS)])
    pltpu.sync_copy(dst_hbm.at[w], dst_v)
    pltpu.sync_copy(ones_hbm, ones_v)
    plsc.subcore_barrier()

    def step(j, carry):
        pltpu.sync_copy(ones_v, acc_sh.at[dst_v.at[j]], add=True)
        return carry

    lax.fori_loop(0, _NCHUNK, step, 0)
    plsc.subcore_barrier()
    pltpu.sync_copy(acc_sh.at[pl.ds(s * _ZROWS, _ZROWS)],
                    outc_hbm.at[c, pl.ds(s * _ZROWS, _ZROWS)])


def _sc_cnt(dst3, zeros, ones):
    mesh = plsc.VectorSubcoreMesh(core_axis_name="c", subcore_axis_name="s")
    return pl.kernel(
        _sc_cnt_body,
        out_type=jax.ShapeDtypeStruct((_NC, _NPAD, _F), jnp.float32),
        mesh=mesh,
        scratch_types=[
            pltpu.VMEM((_NCHUNK, _C), jnp.int32),
            pltpu.VMEM((_C, _F), jnp.float32),
            pltpu.VMEM_SHARED((_NPAD, _F), jnp.float32),
        ],
    )(dst3, zeros, ones)


def _tc1_body(p_ref, cnt_ref, x_ref, w1l_ref, b1l_ref, w1r_ref, g_ref, b_ref,
              h_ref):
    p = p_ref[0] + p_ref[1]
    cnt = (cnt_ref[0] + cnt_ref[1])[:_N]
    mean = p[:_N] / jnp.maximum(cnt, 1.0)
    out = (jnp.dot(mean, w1l_ref[...], preferred_element_type=jnp.float32)
           + b1l_ref[...]
           + jnp.dot(x_ref[...], w1r_ref[...], preferred_element_type=jnp.float32))
    nrm = jnp.sqrt(jnp.sum(out * out, axis=-1, keepdims=True))
    out = out / jnp.maximum(nrm, 1e-12)
    h = jnp.maximum(out, 0.0)
    mu = jnp.mean(h, axis=0, keepdims=True)
    var = jnp.mean((h - mu) ** 2, axis=0, keepdims=True)
    h_ref[...] = g_ref[...] * (h - mu) / jnp.sqrt(var + 1e-5) + b_ref[...]


def _tc_layer1(p, cnt, x, w1l, b1l, w1r, gamma, beta):
    return pl.pallas_call(
        _tc1_body,
        out_shape=jax.ShapeDtypeStruct((_N, _F), jnp.float32),
    )(p, cnt, x, w1l, b1l, w1r, gamma, beta)


def _tc2_body(p_ref, cnt_ref, h_ref, w2l_ref, b2l_ref, w2r_ref, o_ref):
    p = p_ref[0] + p_ref[1]
    cnt = (cnt_ref[0] + cnt_ref[1])[:_N]
    mean = p[:_N] / jnp.maximum(cnt, 1.0)
    out = (jnp.dot(mean, w2l_ref[...], preferred_element_type=jnp.float32)
           + b2l_ref[...]
           + jnp.dot(h_ref[...], w2r_ref[...], preferred_element_type=jnp.float32))
    nrm = jnp.sqrt(jnp.sum(out * out, axis=-1, keepdims=True))
    o_ref[...] = out / jnp.maximum(nrm, 1e-12)


def _tc_layer2(p, cnt, h, w2l, b2l, w2r):
    return pl.pallas_call(
        _tc2_body,
        out_shape=jax.ShapeDtypeStruct((_N, _F), jnp.float32),
    )(p, cnt, h, w2l, b2l, w2r)


def kernel(x, edge_index, W1l, b1l, W1r, W2l, b2l, W2r, gamma, beta):
    src3 = edge_index[0].reshape(_NW, _EW)
    dst3 = edge_index[1].reshape(_NW, _NCHUNK, _C)
    zeros = jnp.zeros((_ZROWS, _F), jnp.float32)
    ones = jnp.ones((_C, _F), jnp.float32)

    cnt_p = _sc_cnt(dst3, zeros, ones)
    p1 = _sc_agg(x, src3, dst3, zeros)
    h = _tc_layer1(p1, cnt_p, x, W1l, b1l.reshape(1, _F), W1r,
                   gamma.reshape(1, _F), beta.reshape(1, _F))
    p2 = _sc_agg(h, src3, dst3, zeros)
    return _tc_layer2(p2, cnt_p, h, W2l, b2l.reshape(1, _F), W2r)


# speedup vs baseline: 1.1818x; 1.1818x over previous
import jax
import jax.numpy as jnp
from jax import lax
from jax.experimental import pallas as pl
from jax.experimental.pallas import tpu as pltpu
from jax.experimental.pallas import tpu_sc as plsc

_N = 10000
_E = 320000
_F = 128
_NC = 2
_NS = 16
_NW = _NC * _NS
_EW = _E // _NW
_C = 80
_NCHUNK = _EW // _C
_NPAD = 10240
_ZROWS = _NPAD // _NS


def _sc_agg_body(src_hbm, dst_hbm, xa_hbm, zero_hbm, out_hbm,
                 src_v, dst_v, rows0, rows1, sem0, sem1, ssem0, ssem1, acc_sh):
    c = lax.axis_index("c")
    s = lax.axis_index("s")
    w = c * _NS + s

    pltpu.sync_copy(zero_hbm, acc_sh.at[pl.ds(s * _ZROWS, _ZROWS)])
    pltpu.sync_copy(src_hbm.at[w], src_v)
    pltpu.sync_copy(dst_hbm.at[w], dst_v)
    plsc.subcore_barrier()

    pltpu.async_copy(xa_hbm.at[src_v.at[pl.ds(0, _C)]], rows0, sem0)
    pltpu.async_copy(xa_hbm.at[src_v.at[pl.ds(_C, _C)]], rows1, sem1)

    def step2(jj, carry):
        j0 = jj * 2
        pltpu.make_async_copy(xa_hbm.at[src_v.at[pl.ds(j0 * _C, _C)]], rows0, sem0).wait()
        pltpu.sync_copy(rows0, acc_sh.at[dst_v.at[j0]], add=True)
        pltpu.async_copy(xa_hbm.at[src_v.at[pl.ds((j0 + 2) * _C, _C)]], rows0, sem0)
        pltpu.make_async_copy(xa_hbm.at[src_v.at[pl.ds((j0 + 1) * _C, _C)]], rows1, sem1).wait()
        pltpu.sync_copy(rows1, acc_sh.at[dst_v.at[j0 + 1]], add=True)

        @pl.when(jj < _NCHUNK // 2 - 1)
        def _():
            pltpu.async_copy(xa_hbm.at[src_v.at[pl.ds((j0 + 3) * _C, _C)]], rows1, sem1)

        return carry

    lax.fori_loop(0, _NCHUNK // 2, step2, 0)
    pltpu.make_async_copy(xa_hbm.at[src_v.at[pl.ds((_NCHUNK - 1) * _C, _C)]], rows0, sem0).wait()
    pltpu.sync_copy(rows0, acc_sh.at[dst_v.at[_NCHUNK - 1]], add=True)

    plsc.subcore_barrier()
    pltpu.sync_copy(acc_sh.at[pl.ds(s * _ZROWS, _ZROWS)],
                    out_hbm.at[c, pl.ds(s * _ZROWS, _ZROWS)])


def _sc_agg(xa, src3, dst3, zeros):
    mesh = plsc.VectorSubcoreMesh(core_axis_name="c", subcore_axis_name="s")
    return pl.kernel(
        _sc_agg_body,
        out_type=jax.ShapeDtypeStruct((_NC, _NPAD, _F), jnp.float32),
        mesh=mesh,
        scratch_types=[
            pltpu.VMEM((_EW,), jnp.int32),
            pltpu.VMEM((_NCHUNK, _C), jnp.int32),
            pltpu.VMEM((_C, _F), jnp.float32),
            pltpu.VMEM((_C, _F), jnp.float32),
            pltpu.SemaphoreType.DMA,
            pltpu.SemaphoreType.DMA,
            pltpu.SemaphoreType.DMA,
            pltpu.SemaphoreType.DMA,
            pltpu.VMEM_SHARED((_NPAD, _F), jnp.float32),
        ],
    )(src3, dst3, xa, zeros)


def _sc_cnt_body(dst_hbm, zero_hbm, ones_hbm, outc_hbm,
                 dst_v, ones_v, acc_sh):
    c = lax.axis_index("c")
    s = lax.axis_index("s")
    w = c * _NS + s

    pltpu.sync_copy(zero_hbm, acc_sh.at[pl.ds(s * _ZROWS, _ZROWS)])
    pltpu.sync_copy(dst_hbm.at[w], dst_v)
    pltpu.sync_copy(ones_hbm, ones_v)
    plsc.subcore_barrier()

    def step(j, carry):
        pltpu.sync_copy(ones_v, acc_sh.at[dst_v.at[j]], add=True)
        return carry

    lax.fori_loop(0, _NCHUNK, step, 0)
    plsc.subcore_barrier()
    pltpu.sync_copy(acc_sh.at[pl.ds(s * _ZROWS, _ZROWS)],
                    outc_hbm.at[c, pl.ds(s * _ZROWS, _ZROWS)])


def _sc_cnt(dst3, zeros, ones):
    mesh = plsc.VectorSubcoreMesh(core_axis_name="c", subcore_axis_name="s")
    return pl.kernel(
        _sc_cnt_body,
        out_type=jax.ShapeDtypeStruct((_NC, _NPAD, _F), jnp.float32),
        mesh=mesh,
        scratch_types=[
            pltpu.VMEM((_NCHUNK, _C), jnp.int32),
            pltpu.VMEM((_C, _F), jnp.float32),
            pltpu.VMEM_SHARED((_NPAD, _F), jnp.float32),
        ],
    )(dst3, zeros, ones)


def _tc1_body(p_ref, cnt_ref, x_ref, w1l_ref, b1l_ref, w1r_ref, g_ref, b_ref,
              h_ref):
    p = p_ref[0] + p_ref[1]
    cnt = (cnt_ref[0] + cnt_ref[1])[:_N]
    mean = p[:_N] / jnp.maximum(cnt, 1.0)
    out = (jnp.dot(mean, w1l_ref[...], preferred_element_type=jnp.float32)
           + b1l_ref[...]
           + jnp.dot(x_ref[...], w1r_ref[...], preferred_element_type=jnp.float32))
    nrm = jnp.sqrt(jnp.sum(out * out, axis=-1, keepdims=True))
    out = out / jnp.maximum(nrm, 1e-12)
    h = jnp.maximum(out, 0.0)
    mu = jnp.mean(h, axis=0, keepdims=True)
    var = jnp.mean((h - mu) ** 2, axis=0, keepdims=True)
    h_ref[...] = g_ref[...] * (h - mu) / jnp.sqrt(var + 1e-5) + b_ref[...]


def _tc_layer1(p, cnt, x, w1l, b1l, w1r, gamma, beta):
    return pl.pallas_call(
        _tc1_body,
        out_shape=jax.ShapeDtypeStruct((_N, _F), jnp.float32),
    )(p, cnt, x, w1l, b1l, w1r, gamma, beta)


def _tc2_body(p_ref, cnt_ref, h_ref, w2l_ref, b2l_ref, w2r_ref, o_ref):
    p = p_ref[0] + p_ref[1]
    cnt = (cnt_ref[0] + cnt_ref[1])[:_N]
    mean = p[:_N] / jnp.maximum(cnt, 1.0)
    out = (jnp.dot(mean, w2l_ref[...], preferred_element_type=jnp.float32)
           + b2l_ref[...]
           + jnp.dot(h_ref[...], w2r_ref[...], preferred_element_type=jnp.float32))
    nrm = jnp.sqrt(jnp.sum(out * out, axis=-1, keepdims=True))
    o_ref[...] = out / jnp.maximum(nrm, 1e-12)


def _tc_layer2(p, cnt, h, w2l, b2l, w2r):
    return pl.pallas_call(
        _tc2_body,
        out_shape=jax.ShapeDtypeStruct((_N, _F), jnp.float32),
    )(p, cnt, h, w2l, b2l, w2r)


def kernel(x, edge_index, W1l, b1l, W1r, W2l, b2l, W2r, gamma, beta):
    src3 = edge_index[0].reshape(_NW, _EW)
    dst3 = edge_index[1].reshape(_NW, _NCHUNK, _C)
    zeros = jnp.zeros((_ZROWS, _F), jnp.float32)
    ones = jnp.ones((_C, _F), jnp.float32)

    cnt_p = _sc_cnt(dst3, zeros, ones)
    p1 = _sc_agg(x, src3, dst3, zeros)
    h = _tc_layer1(p1, cnt_p, x, W1l, b1l.reshape(1, _F), W1r,
                   gamma.reshape(1, _F), beta.reshape(1, _F))
    p2 = _sc_agg(h, src3, dst3, zeros)
    return _tc_layer2(p2, cnt_p, h, W2l, b2l.reshape(1, _F), W2r)


# speedup vs baseline: 1.1930x; 1.0095x over previous
import functools

import jax
import jax.numpy as jnp
from jax import lax
from jax.experimental import pallas as pl
from jax.experimental.pallas import tpu as pltpu
from jax.experimental.pallas import tpu_sc as plsc

_N = 10000
_E = 320000
_F = 128
_NC = 2
_NS = 16
_NW = _NC * _NS
_EW = _E // _NW
_C = 80
_NCHUNK = _EW // _C
_NPAD = 10240
_ZROWS = _NPAD // _NS


def _sc_agg_body(with_counts, *refs):
    if with_counts:
        (src_hbm, dst_hbm, xa_hbm, zero_hbm, ones_hbm, out_hbm, outc_hbm,
         src_v, dst_v, rows0, rows1, sem0, sem1, acc_sh) = refs
    else:
        (src_hbm, dst_hbm, xa_hbm, zero_hbm, out_hbm,
         src_v, dst_v, rows0, rows1, sem0, sem1, acc_sh) = refs
    c = lax.axis_index("c")
    s = lax.axis_index("s")
    w = c * _NS + s

    pltpu.sync_copy(zero_hbm, acc_sh.at[pl.ds(s * _ZROWS, _ZROWS)])
    pltpu.sync_copy(src_hbm.at[w], src_v)
    pltpu.sync_copy(dst_hbm.at[w], dst_v)
    plsc.subcore_barrier()

    pltpu.async_copy(xa_hbm.at[src_v.at[pl.ds(0, _C)]], rows0, sem0)
    pltpu.async_copy(xa_hbm.at[src_v.at[pl.ds(_C, _C)]], rows1, sem1)

    def step2(jj, carry):
        j0 = jj * 2
        pltpu.make_async_copy(xa_hbm.at[src_v.at[pl.ds(j0 * _C, _C)]], rows0, sem0).wait()
        pltpu.sync_copy(rows0, acc_sh.at[dst_v.at[j0]], add=True)
        pltpu.async_copy(xa_hbm.at[src_v.at[pl.ds((j0 + 2) * _C, _C)]], rows0, sem0)
        pltpu.make_async_copy(xa_hbm.at[src_v.at[pl.ds((j0 + 1) * _C, _C)]], rows1, sem1).wait()
        pltpu.sync_copy(rows1, acc_sh.at[dst_v.at[j0 + 1]], add=True)

        @pl.when(jj < _NCHUNK // 2 - 1)
        def _():
            pltpu.async_copy(xa_hbm.at[src_v.at[pl.ds((j0 + 3) * _C, _C)]], rows1, sem1)

        return carry

    lax.fori_loop(0, _NCHUNK // 2, step2, 0)
    pltpu.make_async_copy(xa_hbm.at[src_v.at[pl.ds((_NCHUNK - 1) * _C, _C)]], rows0, sem0).wait()
    pltpu.sync_copy(rows0, acc_sh.at[dst_v.at[_NCHUNK - 1]], add=True)

    plsc.subcore_barrier()
    pltpu.sync_copy(acc_sh.at[pl.ds(s * _ZROWS, _ZROWS)],
                    out_hbm.at[c, pl.ds(s * _ZROWS, _ZROWS)])

    if with_counts:
        plsc.subcore_barrier()
        pltpu.sync_copy(zero_hbm, acc_sh.at[pl.ds(s * _ZROWS, _ZROWS)])
        pltpu.sync_copy(ones_hbm, rows0)
        plsc.subcore_barrier()

        def cstep(j, carry):
            pltpu.sync_copy(rows0, acc_sh.at[dst_v.at[j]], add=True)
            return carry

        lax.fori_loop(0, _NCHUNK, cstep, 0)
        plsc.subcore_barrier()
        pltpu.sync_copy(acc_sh.at[pl.ds(s * _ZROWS, _ZROWS)],
                        outc_hbm.at[c, pl.ds(s * _ZROWS, _ZROWS)])


def _sc_agg(xa, src2, dst3, zeros, ones):
    with_counts = ones is not None
    mesh = plsc.VectorSubcoreMesh(core_axis_name="c", subcore_axis_name="s")
    feat_type = jax.ShapeDtypeStruct((_NC, _NPAD, _F), jnp.float32)
    out_type = (feat_type, feat_type) if with_counts else feat_type
    fn = pl.kernel(
        functools.partial(_sc_agg_body, with_counts),
        out_type=out_type,
        mesh=mesh,
        scratch_types=[
            pltpu.VMEM((_EW,), jnp.int32),
            pltpu.VMEM((_NCHUNK, _C), jnp.int32),
            pltpu.VMEM((_C, _F), jnp.float32),
            pltpu.VMEM((_C, _F), jnp.float32),
            pltpu.SemaphoreType.DMA,
            pltpu.SemaphoreType.DMA,
            pltpu.VMEM_SHARED((_NPAD, _F), jnp.float32),
        ],
    )
    if with_counts:
        return fn(src2, dst3, xa, zeros, ones)
    return fn(src2, dst3, xa, zeros)


def _tc1_body(p_ref, cnt_ref, x_ref, w1l_ref, b1l_ref, w1r_ref, g_ref, b_ref,
              h_ref):
    p = p_ref[0] + p_ref[1]
    cnt = (cnt_ref[0] + cnt_ref[1])[:_N]
    mean = p[:_N] / jnp.maximum(cnt, 1.0)
    out = (jnp.dot(mean, w1l_ref[...], preferred_element_type=jnp.float32)
           + b1l_ref[...]
           + jnp.dot(x_ref[...], w1r_ref[...], preferred_element_type=jnp.float32))
    nrm = jnp.sqrt(jnp.sum(out * out, axis=-1, keepdims=True))
    out = out / jnp.maximum(nrm, 1e-12)
    h = jnp.maximum(out, 0.0)
    mu = jnp.mean(h, axis=0, keepdims=True)
    var = jnp.mean((h - mu) ** 2, axis=0, keepdims=True)
    h_ref[...] = g_ref[...] * (h - mu) / jnp.sqrt(var + 1e-5) + b_ref[...]


def _tc_layer1(p, cnt, x, w1l, b1l, w1r, gamma, beta):
    return pl.pallas_call(
        _tc1_body,
        out_shape=jax.ShapeDtypeStruct((_N, _F), jnp.float32),
    )(p, cnt, x, w1l, b1l, w1r, gamma, beta)


def _tc2_body(p_ref, cnt_ref, h_ref, w2l_ref, b2l_ref, w2r_ref, o_ref):
    p = p_ref[0] + p_ref[1]
    cnt = (cnt_ref[0] + cnt_ref[1])[:_N]
    mean = p[:_N] / jnp.maximum(cnt, 1.0)
    out = (jnp.dot(mean, w2l_ref[...], preferred_element_type=jnp.float32)
           + b2l_ref[...]
           + jnp.dot(h_ref[...], w2r_ref[...], preferred_element_type=jnp.float32))
    nrm = jnp.sqrt(jnp.sum(out * out, axis=-1, keepdims=True))
    o_ref[...] = out / jnp.maximum(nrm, 1e-12)


def _tc_layer2(p, cnt, h, w2l, b2l, w2r):
    return pl.pallas_call(
        _tc2_body,
        out_shape=jax.ShapeDtypeStruct((_N, _F), jnp.float32),
    )(p, cnt, h, w2l, b2l, w2r)


def kernel(x, edge_index, W1l, b1l, W1r, W2l, b2l, W2r, gamma, beta):
    src3 = edge_index[0].reshape(_NW, _EW)
    dst3 = edge_index[1].reshape(_NW, _NCHUNK, _C)
    zeros = jnp.zeros((_ZROWS, _F), jnp.float32)
    ones = jnp.ones((_C, _F), jnp.float32)

    p1, cnt_p = _sc_agg(x, src3, dst3, zeros, ones)
    h = _tc_layer1(p1, cnt_p, x, W1l, b1l.reshape(1, _F), W1r,
                   gamma.reshape(1, _F), beta.reshape(1, _F))
    p2 = _sc_agg(h, src3, dst3, zeros, None)
    return _tc_layer2(p2, cnt_p, h, W2l, b2l.reshape(1, _F), W2r)


# speedup vs baseline: 1.1993x; 1.0053x over previous
import functools

import jax
import jax.numpy as jnp
from jax import lax
from jax.experimental import pallas as pl
from jax.experimental.pallas import tpu as pltpu
from jax.experimental.pallas import tpu_sc as plsc

_N = 10000
_E = 320000
_F = 128
_NC = 2
_NS = 16
_NW = _NC * _NS
_EW = _E // _NW
_C = 80
_NCHUNK = _EW // _C
_NPAD = 10240
_ZROWS = _NPAD // _NS


def _sc_agg_body(with_counts, *refs):
    if with_counts:
        (src_hbm, dst_hbm, xa_hbm, zero_hbm, ones_hbm, out_hbm, outc_hbm,
         src_v, dst_v, rows0, rows1, sem0, sem1, acc_sh) = refs
    else:
        (src_hbm, dst_hbm, xa_hbm, zero_hbm, out_hbm,
         src_v, dst_v, rows0, rows1, sem0, sem1, acc_sh) = refs
    c = lax.axis_index("c")
    s = lax.axis_index("s")
    w = c * _NS + s

    pltpu.sync_copy(src_hbm.at[w], src_v)
    pltpu.async_copy(xa_hbm.at[src_v.at[pl.ds(0, _C)]], rows0, sem0)
    pltpu.async_copy(xa_hbm.at[src_v.at[pl.ds(_C, _C)]], rows1, sem1)
    pltpu.sync_copy(zero_hbm, acc_sh.at[pl.ds(s * _ZROWS, _ZROWS)])
    pltpu.sync_copy(dst_hbm.at[w], dst_v)
    plsc.subcore_barrier()

    def step2(jj, carry):
        j0 = jj * 2
        pltpu.make_async_copy(xa_hbm.at[src_v.at[pl.ds(j0 * _C, _C)]], rows0, sem0).wait()
        pltpu.sync_copy(rows0, acc_sh.at[dst_v.at[j0]], add=True)
        pltpu.async_copy(xa_hbm.at[src_v.at[pl.ds((j0 + 2) * _C, _C)]], rows0, sem0)
        pltpu.make_async_copy(xa_hbm.at[src_v.at[pl.ds((j0 + 1) * _C, _C)]], rows1, sem1).wait()
        pltpu.sync_copy(rows1, acc_sh.at[dst_v.at[j0 + 1]], add=True)

        @pl.when(jj < _NCHUNK // 2 - 1)
        def _():
            pltpu.async_copy(xa_hbm.at[src_v.at[pl.ds((j0 + 3) * _C, _C)]], rows1, sem1)

        return carry

    lax.fori_loop(0, _NCHUNK // 2, step2, 0)
    pltpu.make_async_copy(xa_hbm.at[src_v.at[pl.ds((_NCHUNK - 1) * _C, _C)]], rows0, sem0).wait()
    pltpu.sync_copy(rows0, acc_sh.at[dst_v.at[_NCHUNK - 1]], add=True)

    plsc.subcore_barrier()
    pltpu.sync_copy(acc_sh.at[pl.ds(s * _ZROWS, _ZROWS)],
                    out_hbm.at[c, pl.ds(s * _ZROWS, _ZROWS)])

    if with_counts:
        plsc.subcore_barrier()
        pltpu.sync_copy(zero_hbm, acc_sh.at[pl.ds(s * _ZROWS, _ZROWS)])
        pltpu.sync_copy(ones_hbm, rows0)
        plsc.subcore_barrier()

        def cstep(j, carry):
            pltpu.sync_copy(rows0, acc_sh.at[dst_v.at[j]], add=True)
            return carry

        lax.fori_loop(0, _NCHUNK, cstep, 0)
        plsc.subcore_barrier()
        pltpu.sync_copy(acc_sh.at[pl.ds(s * _ZROWS, _ZROWS)],
                        outc_hbm.at[c, pl.ds(s * _ZROWS, _ZROWS)])


def _sc_agg(xa, src2, dst3, zeros, ones):
    with_counts = ones is not None
    mesh = plsc.VectorSubcoreMesh(core_axis_name="c", subcore_axis_name="s")
    feat_type = jax.ShapeDtypeStruct((_NC, _NPAD, _F), jnp.float32)
    out_type = (feat_type, feat_type) if with_counts else feat_type
    fn = pl.kernel(
        functools.partial(_sc_agg_body, with_counts),
        out_type=out_type,
        mesh=mesh,
        scratch_types=[
            pltpu.VMEM((_EW,), jnp.int32),
            pltpu.VMEM((_NCHUNK, _C), jnp.int32),
            pltpu.VMEM((_C, _F), jnp.float32),
            pltpu.VMEM((_C, _F), jnp.float32),
            pltpu.SemaphoreType.DMA,
            pltpu.SemaphoreType.DMA,
            pltpu.VMEM_SHARED((_NPAD, _F), jnp.float32),
        ],
    )
    if with_counts:
        return fn(src2, dst3, xa, zeros, ones)
    return fn(src2, dst3, xa, zeros)


def _tc1_body(p_ref, cnt_ref, x_ref, w1l_ref, b1l_ref, w1r_ref, g_ref, b_ref,
              h_ref):
    p = p_ref[0] + p_ref[1]
    cnt = (cnt_ref[0] + cnt_ref[1])[:_N]
    mean = p[:_N] / jnp.maximum(cnt, 1.0)
    out = (jnp.dot(mean, w1l_ref[...], preferred_element_type=jnp.float32)
           + b1l_ref[...]
           + jnp.dot(x_ref[...], w1r_ref[...], preferred_element_type=jnp.float32))
    nrm = jnp.sqrt(jnp.sum(out * out, axis=-1, keepdims=True))
    out = out / jnp.maximum(nrm, 1e-12)
    h = jnp.maximum(out, 0.0)
    mu = jnp.mean(h, axis=0, keepdims=True)
    var = jnp.mean((h - mu) ** 2, axis=0, keepdims=True)
    h_ref[...] = g_ref[...] * (h - mu) / jnp.sqrt(var + 1e-5) + b_ref[...]


def _tc_layer1(p, cnt, x, w1l, b1l, w1r, gamma, beta):
    return pl.pallas_call(
        _tc1_body,
        out_shape=jax.ShapeDtypeStruct((_N, _F), jnp.float32),
    )(p, cnt, x, w1l, b1l, w1r, gamma, beta)


def _tc2_body(p_ref, cnt_ref, h_ref, w2l_ref, b2l_ref, w2r_ref, o_ref):
    p = p_ref[0] + p_ref[1]
    cnt = (cnt_ref[0] + cnt_ref[1])[:_N]
    mean = p[:_N] / jnp.maximum(cnt, 1.0)
    out = (jnp.dot(mean, w2l_ref[...], preferred_element_type=jnp.float32)
           + b2l_ref[...]
           + jnp.dot(h_ref[...], w2r_ref[...], preferred_element_type=jnp.float32))
    nrm = jnp.sqrt(jnp.sum(out * out, axis=-1, keepdims=True))
    o_ref[...] = out / jnp.maximum(nrm, 1e-12)


def _tc_layer2(p, cnt, h, w2l, b2l, w2r):
    return pl.pallas_call(
        _tc2_body,
        out_shape=jax.ShapeDtypeStruct((_N, _F), jnp.float32),
    )(p, cnt, h, w2l, b2l, w2r)


def kernel(x, edge_index, W1l, b1l, W1r, W2l, b2l, W2r, gamma, beta):
    src3 = edge_index[0].reshape(_NW, _EW)
    dst3 = edge_index[1].reshape(_NW, _NCHUNK, _C)
    zeros = jnp.zeros((_ZROWS, _F), jnp.float32)
    ones = jnp.ones((_C, _F), jnp.float32)

    p1, cnt_p = _sc_agg(x, src3, dst3, zeros, ones)
    h = _tc_layer1(p1, cnt_p, x, W1l, b1l.reshape(1, _F), W1r,
                   gamma.reshape(1, _F), beta.reshape(1, _F))
    p2 = _sc_agg(h, src3, dst3, zeros, None)
    return _tc_layer2(p2, cnt_p, h, W2l, b2l.reshape(1, _F), W2r)
